# Initial kernel scaffold; baseline (speedup 1.0000x reference)
import jax
import jax.numpy as jnp
from jax.experimental import pallas as pl


def kernel(xt, predicted_noise, time_steps, prev_time_steps):
    raise NotImplementedError("write your pallas kernel here")


# speedup vs baseline: 3.5825x; 3.5825x over previous
import jax
import jax.numpy as jnp
from jax.experimental import pallas as pl
from jax.experimental.pallas import tpu as pltpu

_TAU_NUM_STEPS = 50
_NUM_TRAIN_STEPS = 1000


def _tau_tables():
    betas = jnp.linspace(1e-4, 0.02, _NUM_TRAIN_STEPS, dtype=jnp.float32)
    alphas = 1.0 - betas
    alpha_bars = jnp.cumprod(alphas)
    tau = jnp.linspace(0, _NUM_TRAIN_STEPS - 1, _TAU_NUM_STEPS).astype(jnp.int32)
    tau_alpha_bars = jnp.take(alpha_bars, tau)
    sac = jnp.sqrt(tau_alpha_bars)
    somac = jnp.sqrt(1.0 - tau_alpha_bars)
    return sac, somac


def _body(ts_ref, pts_ref, sac_ref, somac_ref, xt_ref, pn_ref, xtp_ref, x0_ref):
    i = pl.program_id(0)
    t = ts_ref[i]
    p = pts_ref[i]
    sac = sac_ref[t]
    somac = somac_ref[t]
    psac = sac_ref[p]
    psomac = somac_ref[p]
    dc = jnp.sqrt(jnp.maximum(psomac * psomac, jnp.float32(1e-8)))
    rsac = 1.0 / sac
    pn = pn_ref[...]
    x0 = (xt_ref[...] - somac * pn) * rsac
    x0_ref[...] = x0
    xtp_ref[...] = psac * x0 + dc * pn


def kernel(xt, predicted_noise, time_steps, prev_time_steps):
    B, C, H, W = xt.shape
    rows = C * H * W // W
    sac, somac = _tau_tables()
    x3 = xt.reshape(B, rows, W)
    p3 = predicted_noise.reshape(B, rows, W)
    grid_spec = pltpu.PrefetchScalarGridSpec(
        num_scalar_prefetch=4,
        grid=(B,),
        in_specs=[
            pl.BlockSpec((1, rows, W), lambda i, *_: (i, 0, 0)),
            pl.BlockSpec((1, rows, W), lambda i, *_: (i, 0, 0)),
        ],
        out_specs=[
            pl.BlockSpec((1, rows, W), lambda i, *_: (i, 0, 0)),
            pl.BlockSpec((1, rows, W), lambda i, *_: (i, 0, 0)),
        ],
    )
    xtp, x0 = pl.pallas_call(
        _body,
        grid_spec=grid_spec,
        out_shape=[jax.ShapeDtypeStruct((B, rows, W), jnp.float32)] * 2,
    )(
        time_steps.astype(jnp.int32),
        prev_time_steps.astype(jnp.int32),
        sac,
        somac,
        x3,
        p3,
    )
    return xtp.reshape(B, C, H, W), x0.reshape(B, C, H, W)
